# Initial kernel scaffold; baseline (speedup 1.0000x reference)
import jax
import jax.numpy as jnp
from jax.experimental import pallas as pl


def kernel(output, target):
    raise NotImplementedError("write your pallas kernel here")


# speedup vs baseline: 2.1055x; 2.1055x over previous
import math

import jax
import jax.numpy as jnp
from jax.experimental import pallas as pl

_B = 1024
_V = 100000
_LS = 0.1
_PAD = 0
_CONF = 1.0 - _LS
_SMOOTH = _LS / (_V - 2)
_CONST = _LS * math.log(_SMOOTH) + _CONF * math.log(_CONF)

_BB = 8


def _loss_kernel(tgt_ref, out_ref, loss_ref):
    x = out_ref[...]
    t = tgt_ref[...]
    row_sum = jnp.sum(x, axis=1, keepdims=True)
    ids = jax.lax.broadcasted_iota(jnp.int32, x.shape, 1)
    o_t = jnp.sum(jnp.where(ids == t, x, 0.0), axis=1, keepdims=True)
    o_pad = x[:, _PAD:_PAD + 1]
    loss = _SMOOTH * row_sum + (_CONF - _SMOOTH) * o_t - _SMOOTH * o_pad - _CONST
    loss_ref[...] = jnp.where(t == _PAD, 0.0, loss)


def kernel(output, target):
    tgt = target.astype(jnp.int32).reshape(_B, 1)
    loss = pl.pallas_call(
        _loss_kernel,
        grid=(_B // _BB,),
        in_specs=[
            pl.BlockSpec((_BB, 1), lambda i: (i, 0)),
            pl.BlockSpec((_BB, _V), lambda i: (i, 0)),
        ],
        out_specs=pl.BlockSpec((_BB, 1), lambda i: (i, 0)),
        out_shape=jax.ShapeDtypeStruct((_B, 1), jnp.float32),
    )(tgt, output)
    return loss.reshape(_B)


# speedup vs baseline: 2.1249x; 1.0092x over previous
import math

import jax
import jax.numpy as jnp
from jax.experimental import pallas as pl
from jax.experimental.pallas import tpu as pltpu

_B = 1024
_V = 100000
_LS = 0.1
_PAD = 0
_CONF = 1.0 - _LS
_SMOOTH = _LS / (_V - 2)
_CONST = _LS * math.log(_SMOOTH) + _CONF * math.log(_CONF)

_BB = 8


def _loss_kernel(tgt_ref, out_ref, loss_ref):
    i = pl.program_id(0)
    x = out_ref[...]
    row_sum = jnp.sum(x, axis=1, keepdims=True)

    lane = jax.lax.broadcasted_iota(jnp.int32, (1, 128), 1)
    sel_rows = []
    t_rows = []
    for r in range(_BB):
        t_r = tgt_ref[i * _BB + r]
        base = (t_r // 128) * 128
        chunk = out_ref[r:r + 1, pl.ds(base, 128)]
        sel_rows.append(jnp.where(lane == t_r - base, chunk, 0.0))
        t_rows.append(jnp.full((1, 1), t_r, dtype=jnp.int32))
    o_t = jnp.sum(jnp.concatenate(sel_rows, axis=0), axis=1, keepdims=True)
    t_vec = jnp.concatenate(t_rows, axis=0)

    o_pad = x[:, _PAD:_PAD + 1]
    loss = _SMOOTH * row_sum + (_CONF - _SMOOTH) * o_t - _SMOOTH * o_pad - _CONST
    loss_ref[...] = jnp.where(t_vec == _PAD, 0.0, loss)


def kernel(output, target):
    tgt = target.astype(jnp.int32)
    grid_spec = pltpu.PrefetchScalarGridSpec(
        num_scalar_prefetch=1,
        grid=(_B // _BB,),
        in_specs=[pl.BlockSpec((_BB, _V), lambda i, tgt_ref: (i, 0))],
        out_specs=pl.BlockSpec((_BB, 1), lambda i, tgt_ref: (i, 0)),
    )
    loss = pl.pallas_call(
        _loss_kernel,
        grid_spec=grid_spec,
        out_shape=jax.ShapeDtypeStruct((_B, 1), jnp.float32),
    )(tgt, output)
    return loss.reshape(_B)


# speedup vs baseline: 2.4163x; 1.1371x over previous
import math

import jax
import jax.numpy as jnp
from jax.experimental import pallas as pl
from jax.experimental.pallas import tpu as pltpu

_B = 1024
_V = 100000
_LS = 0.1
_PAD = 0
_CONF = 1.0 - _LS
_SMOOTH = _LS / (_V - 2)
_CONST = _LS * math.log(_SMOOTH) + _CONF * math.log(_CONF)

_BB = 32


def _loss_kernel(tgt_ref, out_ref, loss_ref):
    i = pl.program_id(0)
    x = out_ref[...]
    row_sum = jnp.sum(x, axis=1, keepdims=True)

    lane = jax.lax.broadcasted_iota(jnp.int32, (1, 128), 1)
    sel_rows = []
    t_rows = []
    for r in range(_BB):
        t_r = tgt_ref[i * _BB + r]
        base = (t_r // 128) * 128
        chunk = out_ref[r:r + 1, pl.ds(base, 128)]
        sel_rows.append(jnp.where(lane == t_r - base, chunk, 0.0))
        t_rows.append(jnp.full((1, 1), t_r, dtype=jnp.int32))
    o_t = jnp.sum(jnp.concatenate(sel_rows, axis=0), axis=1, keepdims=True)
    t_vec = jnp.concatenate(t_rows, axis=0)

    o_pad = x[:, _PAD:_PAD + 1]
    loss = _SMOOTH * row_sum + (_CONF - _SMOOTH) * o_t - _SMOOTH * o_pad - _CONST
    loss_ref[...] = jnp.where(t_vec == _PAD, 0.0, loss)


def kernel(output, target):
    tgt = target.astype(jnp.int32)
    grid_spec = pltpu.PrefetchScalarGridSpec(
        num_scalar_prefetch=1,
        grid=(_B // _BB,),
        in_specs=[pl.BlockSpec((_BB, _V), lambda i, tgt_ref: (i, 0))],
        out_specs=pl.BlockSpec((_BB, 1), lambda i, tgt_ref: (i, 0)),
    )
    loss = pl.pallas_call(
        _loss_kernel,
        grid_spec=grid_spec,
        out_shape=jax.ShapeDtypeStruct((_B, 1), jnp.float32),
    )(tgt, output)
    return loss.reshape(_B)
